# baseline (speedup 1.0000x reference)
import functools

import jax
import jax.numpy as jnp
from jax import lax
from jax.experimental import pallas as pl
from jax.experimental.pallas import tpu as pltpu
from jax.experimental.pallas import tpu_sc as plsc

_CANVAS = 1024
_NC = 2
_NS = 16
_NW = _NC * _NS
_L = 16
_CHUNK = 16384
_NBUF = 2


def _sc_absdiff(a_hbm, b_hbm, out_hbm, a_buf, b_buf, acc_ref, sems_a, sems_b,
                *, total):
    c = lax.axis_index("c")
    s = lax.axis_index("s")
    wid = s * _NC + c
    per_w = total // _NW
    base = wid * per_w
    nchunks = per_w // _CHUNK

    def start(ci, slot):
        off = base + ci * _CHUNK
        pltpu.make_async_copy(
            a_hbm.at[pl.ds(off, _CHUNK)], a_buf.at[slot], sems_a.at[slot]
        ).start()
        pltpu.make_async_copy(
            b_hbm.at[pl.ds(off, _CHUNK)], b_buf.at[slot], sems_b.at[slot]
        ).start()

    def wait(ci, slot):
        off = base + ci * _CHUNK
        pltpu.make_async_copy(
            a_hbm.at[pl.ds(off, _CHUNK)], a_buf.at[slot], sems_a.at[slot]
        ).wait()
        pltpu.make_async_copy(
            b_hbm.at[pl.ds(off, _CHUNK)], b_buf.at[slot], sems_b.at[slot]
        ).wait()

    for p in range(_NBUF):
        start(p, p)

    acc = jnp.zeros((_L,), dtype=jnp.float32)
    for ci in range(nchunks):
        slot = ci % _NBUF
        wait(ci, slot)

        def body(j, a, _slot=slot):
            off = j * (4 * _L)
            for u in range(4):
                av = a_buf[_slot, pl.ds(off + u * _L, _L)]
                bv = b_buf[_slot, pl.ds(off + u * _L, _L)]
                a = a + jnp.abs(av - bv)
            return a

        acc = lax.fori_loop(0, _CHUNK // (4 * _L), body, acc)
        if ci + _NBUF < nchunks:
            start(ci + _NBUF, slot)

    acc_ref[...] = acc
    pltpu.sync_copy(acc_ref, out_hbm.at[wid])


def kernel(sr, hr, patch_cord, h_idx, w_idx):
    b, c, ph, pw = sr.shape
    scale = 1.0 / (b * c * _CANVAS * _CANVAS)
    total = b * c * ph * pw
    a1 = sr.reshape(total)
    b1 = hr.reshape(total)

    mesh = plsc.VectorSubcoreMesh(core_axis_name="c", subcore_axis_name="s")
    sc_fn = functools.partial(
        pl.kernel,
        mesh=mesh,
        out_type=jax.ShapeDtypeStruct((_NW, _L), jnp.float32),
        scratch_types=[
            pltpu.VMEM((_NBUF, _CHUNK), jnp.float32),
            pltpu.VMEM((_NBUF, _CHUNK), jnp.float32),
            pltpu.VMEM((_L,), jnp.float32),
            pltpu.SemaphoreType.DMA((_NBUF,)),
            pltpu.SemaphoreType.DMA((_NBUF,)),
        ],
    )(functools.partial(_sc_absdiff, total=total))
    partials = sc_fn(a1, b1)
    return jnp.sum(partials) * scale


# speedup vs baseline: 3.1825x; 3.1825x over previous
import functools

import jax
import jax.numpy as jnp
from jax import lax
from jax.experimental import pallas as pl
from jax.experimental.pallas import tpu as pltpu
from jax.experimental.pallas import tpu_sc as plsc

_CANVAS = 1024

_TC_CHUNK_ROWS = 1024
_TC_NBUF = 4


def _tc_absdiff(a_hbm, b_hbm, out_ref, a_buf, b_buf, a_sem, b_sem, *, nchunks):
    def start(i, slot):
        rows = pl.ds(i * _TC_CHUNK_ROWS, _TC_CHUNK_ROWS)
        pltpu.make_async_copy(a_hbm.at[rows, :], a_buf.at[slot], a_sem.at[slot]).start()
        pltpu.make_async_copy(b_hbm.at[rows, :], b_buf.at[slot], b_sem.at[slot]).start()

    def wait(i, slot):
        rows = pl.ds(i * _TC_CHUNK_ROWS, _TC_CHUNK_ROWS)
        pltpu.make_async_copy(a_hbm.at[rows, :], a_buf.at[slot], a_sem.at[slot]).wait()
        pltpu.make_async_copy(b_hbm.at[rows, :], b_buf.at[slot], b_sem.at[slot]).wait()

    for s in range(_TC_NBUF):
        start(s, s)

    acc = jnp.zeros((8, 128), dtype=jnp.float32)
    for i in range(nchunks):
        slot = i % _TC_NBUF
        wait(i, slot)
        d = jnp.abs(a_buf[slot] - b_buf[slot])
        acc += jnp.sum(d.reshape(-1, 8, 128), axis=0)
        if i + _TC_NBUF < nchunks:
            start(i + _TC_NBUF, slot)

    out_ref[0, 0] = jnp.sum(acc)


_NC = 2
_NS = 16
_NW = _NC * _NS
_L = 16
_SC_CH_ROWS = 48
_SC_NBUF = 2


def _sc_absdiff(a_hbm, b_hbm, out_hbm, a_buf, b_buf, acc_ref, sems_a, sems_b,
                *, row0, rows_per_w):
    c = lax.axis_index("c")
    s = lax.axis_index("s")
    wid = s * _NC + c
    base = row0 + wid * rows_per_w
    nchunks = rows_per_w // _SC_CH_ROWS

    def start(ci, slot):
        rows = pl.ds(base + ci * _SC_CH_ROWS, _SC_CH_ROWS)
        pltpu.make_async_copy(a_hbm.at[rows, :], a_buf.at[slot], sems_a.at[slot]).start()
        pltpu.make_async_copy(b_hbm.at[rows, :], b_buf.at[slot], sems_b.at[slot]).start()

    def wait(ci, slot):
        rows = pl.ds(base + ci * _SC_CH_ROWS, _SC_CH_ROWS)
        pltpu.make_async_copy(a_hbm.at[rows, :], a_buf.at[slot], sems_a.at[slot]).wait()
        pltpu.make_async_copy(b_hbm.at[rows, :], b_buf.at[slot], sems_b.at[slot]).wait()

    for p in range(_SC_NBUF):
        start(p, p)

    accs = [jnp.zeros((_L,), dtype=jnp.float32) for _ in range(4)]
    for ci in range(nchunks):
        slot = ci % _SC_NBUF
        wait(ci, slot)

        def body(j, accs4, _slot=slot):
            a0, a1, a2, a3 = accs4
            outs = [a0, a1, a2, a3]
            for u in range(32):
                col = pl.ds(u * _L, _L)
                av = a_buf[_slot, j, col]
                bv = b_buf[_slot, j, col]
                outs[u % 4] = outs[u % 4] + jnp.abs(av - bv)
            return tuple(outs)

        accs = list(lax.fori_loop(0, _SC_CH_ROWS, body, tuple(accs)))
        if ci + _SC_NBUF < nchunks:
            start(ci + _SC_NBUF, slot)

    acc_ref[...] = (accs[0] + accs[1]) + (accs[2] + accs[3])
    pltpu.sync_copy(acc_ref, out_hbm.at[wid])


def kernel(sr, hr, patch_cord, h_idx, w_idx):
    b, c, ph, pw = sr.shape
    scale = 1.0 / (b * c * _CANVAS * _CANVAS)
    rows = b * c * ph
    a2 = sr.reshape(rows, pw)
    b2 = hr.reshape(rows, pw)

    rows_per_w = 192
    sc_rows = rows_per_w * _NW
    tc_rows = rows - sc_rows
    tc_nchunks = tc_rows // _TC_CHUNK_ROWS
    assert tc_nchunks * _TC_CHUNK_ROWS == tc_rows

    mesh = plsc.VectorSubcoreMesh(core_axis_name="c", subcore_axis_name="s")
    sc_fn = functools.partial(
        pl.kernel,
        mesh=mesh,
        out_type=jax.ShapeDtypeStruct((_NW, _L), jnp.float32),
        scratch_types=[
            pltpu.VMEM((_SC_NBUF, _SC_CH_ROWS, pw), jnp.float32),
            pltpu.VMEM((_SC_NBUF, _SC_CH_ROWS, pw), jnp.float32),
            pltpu.VMEM((_L,), jnp.float32),
            pltpu.SemaphoreType.DMA((_SC_NBUF,)),
            pltpu.SemaphoreType.DMA((_SC_NBUF,)),
        ],
    )(functools.partial(_sc_absdiff, row0=tc_rows, rows_per_w=rows_per_w))
    sc_partials = sc_fn(a2, b2)

    tc_part = pl.pallas_call(
        functools.partial(_tc_absdiff, nchunks=tc_nchunks),
        in_specs=[
            pl.BlockSpec(memory_space=pl.ANY),
            pl.BlockSpec(memory_space=pl.ANY),
        ],
        out_specs=pl.BlockSpec(memory_space=pltpu.SMEM),
        out_shape=jax.ShapeDtypeStruct((1, 1), jnp.float32),
        scratch_shapes=[
            pltpu.VMEM((_TC_NBUF, _TC_CHUNK_ROWS, pw), jnp.float32),
            pltpu.VMEM((_TC_NBUF, _TC_CHUNK_ROWS, pw), jnp.float32),
            pltpu.SemaphoreType.DMA((_TC_NBUF,)),
            pltpu.SemaphoreType.DMA((_TC_NBUF,)),
        ],
    )(a2, b2)

    return (tc_part[0, 0] + jnp.sum(sc_partials)) * scale


# speedup vs baseline: 3.1867x; 1.0013x over previous
import functools

import jax
import jax.numpy as jnp
from jax import lax
from jax.experimental import pallas as pl
from jax.experimental.pallas import tpu as pltpu
from jax.experimental.pallas import tpu_sc as plsc

_CANVAS = 1024

_TC_CHUNK_ROWS = 1024
_TC_NBUF = 4


def _tc_absdiff(a_hbm, b_hbm, out_ref, a_buf, b_buf, a_sem, b_sem, *, nchunks):
    def start(i, slot):
        rows = pl.ds(i * _TC_CHUNK_ROWS, _TC_CHUNK_ROWS)
        pltpu.make_async_copy(a_hbm.at[rows, :], a_buf.at[slot], a_sem.at[slot]).start()
        pltpu.make_async_copy(b_hbm.at[rows, :], b_buf.at[slot], b_sem.at[slot]).start()

    def wait(i, slot):
        rows = pl.ds(i * _TC_CHUNK_ROWS, _TC_CHUNK_ROWS)
        pltpu.make_async_copy(a_hbm.at[rows, :], a_buf.at[slot], a_sem.at[slot]).wait()
        pltpu.make_async_copy(b_hbm.at[rows, :], b_buf.at[slot], b_sem.at[slot]).wait()

    for s in range(_TC_NBUF):
        start(s, s)

    acc = jnp.zeros((8, 128), dtype=jnp.float32)
    for i in range(nchunks):
        slot = i % _TC_NBUF
        wait(i, slot)
        d = jnp.abs(a_buf[slot] - b_buf[slot])
        acc += jnp.sum(d.reshape(-1, 8, 128), axis=0)
        if i + _TC_NBUF < nchunks:
            start(i + _TC_NBUF, slot)

    out_ref[0, 0] = jnp.sum(acc)


_NC = 2
_NS = 16
_NW = _NC * _NS
_L = 16
_SC_CH_ROWS = 48
_SC_NBUF = 2


def _sc_absdiff(a_hbm, b_hbm, out_hbm, a_buf, b_buf, acc_ref, sems_a, sems_b,
                *, row0, rows_per_w):
    c = lax.axis_index("c")
    s = lax.axis_index("s")
    wid = s * _NC + c
    base = row0 + wid * rows_per_w
    nchunks = rows_per_w // _SC_CH_ROWS

    def start(ci, slot):
        rows = pl.ds(base + ci * _SC_CH_ROWS, _SC_CH_ROWS)
        pltpu.make_async_copy(a_hbm.at[rows, :], a_buf.at[slot], sems_a.at[slot]).start()
        pltpu.make_async_copy(b_hbm.at[rows, :], b_buf.at[slot], sems_b.at[slot]).start()

    def wait(ci, slot):
        rows = pl.ds(base + ci * _SC_CH_ROWS, _SC_CH_ROWS)
        pltpu.make_async_copy(a_hbm.at[rows, :], a_buf.at[slot], sems_a.at[slot]).wait()
        pltpu.make_async_copy(b_hbm.at[rows, :], b_buf.at[slot], sems_b.at[slot]).wait()

    for p in range(_SC_NBUF):
        start(p, p)

    accs = [jnp.zeros((_L,), dtype=jnp.float32) for _ in range(4)]
    for ci in range(nchunks):
        slot = ci % _SC_NBUF
        wait(ci, slot)

        def body(j, accs4, _slot=slot):
            a0, a1, a2, a3 = accs4
            outs = [a0, a1, a2, a3]
            for u in range(32):
                col = pl.ds(u * _L, _L)
                av = a_buf[_slot, j, col]
                bv = b_buf[_slot, j, col]
                outs[u % 4] = outs[u % 4] + jnp.abs(av - bv)
            return tuple(outs)

        accs = list(lax.fori_loop(0, _SC_CH_ROWS, body, tuple(accs)))
        if ci + _SC_NBUF < nchunks:
            start(ci + _SC_NBUF, slot)

    acc_ref[...] = (accs[0] + accs[1]) + (accs[2] + accs[3])
    pltpu.sync_copy(acc_ref, out_hbm.at[wid])


def kernel(sr, hr, patch_cord, h_idx, w_idx):
    b, c, ph, pw = sr.shape
    scale = 1.0 / (b * c * _CANVAS * _CANVAS)
    rows = b * c * ph
    a2 = sr.reshape(rows, pw)
    b2 = hr.reshape(rows, pw)

    rows_per_w = 192
    sc_rows = rows_per_w * _NW
    tc_rows = rows - sc_rows
    tc_nchunks = tc_rows // _TC_CHUNK_ROWS
    assert tc_nchunks * _TC_CHUNK_ROWS == tc_rows

    tc_part = pl.pallas_call(
        functools.partial(_tc_absdiff, nchunks=tc_nchunks),
        in_specs=[
            pl.BlockSpec(memory_space=pl.ANY),
            pl.BlockSpec(memory_space=pl.ANY),
        ],
        out_specs=pl.BlockSpec(memory_space=pltpu.SMEM),
        out_shape=jax.ShapeDtypeStruct((1, 1), jnp.float32),
        scratch_shapes=[
            pltpu.VMEM((_TC_NBUF, _TC_CHUNK_ROWS, pw), jnp.float32),
            pltpu.VMEM((_TC_NBUF, _TC_CHUNK_ROWS, pw), jnp.float32),
            pltpu.SemaphoreType.DMA((_TC_NBUF,)),
            pltpu.SemaphoreType.DMA((_TC_NBUF,)),
        ],
    )(a2, b2)

    mesh = plsc.VectorSubcoreMesh(core_axis_name="c", subcore_axis_name="s")
    sc_fn = functools.partial(
        pl.kernel,
        mesh=mesh,
        out_type=jax.ShapeDtypeStruct((_NW, _L), jnp.float32),
        scratch_types=[
            pltpu.VMEM((_SC_NBUF, _SC_CH_ROWS, pw), jnp.float32),
            pltpu.VMEM((_SC_NBUF, _SC_CH_ROWS, pw), jnp.float32),
            pltpu.VMEM((_L,), jnp.float32),
            pltpu.SemaphoreType.DMA((_SC_NBUF,)),
            pltpu.SemaphoreType.DMA((_SC_NBUF,)),
        ],
    )(functools.partial(_sc_absdiff, row0=tc_rows, rows_per_w=rows_per_w))
    sc_partials = sc_fn(a2, b2)

    return (tc_part[0, 0] + jnp.sum(sc_partials)) * scale


# speedup vs baseline: 5.1649x; 1.6208x over previous
import functools

import jax
import jax.numpy as jnp
from jax.experimental import pallas as pl
from jax.experimental.pallas import tpu as pltpu

_CANVAS = 1024
_CHUNK_ROWS = 1024
_NBUF = 6


def _absdiff_manual(a_hbm, b_hbm, out_ref, a_buf, b_buf, a_sem, b_sem, *,
                    scale, nchunks):
    def start(i, slot):
        rows = pl.ds(i * _CHUNK_ROWS, _CHUNK_ROWS)
        pltpu.make_async_copy(a_hbm.at[rows, :], a_buf.at[slot], a_sem.at[slot]).start()
        pltpu.make_async_copy(b_hbm.at[rows, :], b_buf.at[slot], b_sem.at[slot]).start()

    def wait(i, slot):
        rows = pl.ds(i * _CHUNK_ROWS, _CHUNK_ROWS)
        pltpu.make_async_copy(a_hbm.at[rows, :], a_buf.at[slot], a_sem.at[slot]).wait()
        pltpu.make_async_copy(b_hbm.at[rows, :], b_buf.at[slot], b_sem.at[slot]).wait()

    for s in range(_NBUF):
        start(s, s)

    acc = jnp.zeros((8, 128), dtype=jnp.float32)
    for i in range(nchunks):
        slot = i % _NBUF
        wait(i, slot)
        d = jnp.abs(a_buf[slot] - b_buf[slot])
        acc += jnp.sum(d.reshape(-1, 8, 128), axis=0)
        if i + _NBUF < nchunks:
            start(i + _NBUF, slot)

    out_ref[0, 0] = jnp.sum(acc) * scale


def kernel(sr, hr, patch_cord, h_idx, w_idx):
    b, c, ph, pw = sr.shape
    scale = 1.0 / (b * c * _CANVAS * _CANVAS)
    rows = b * c * ph
    nchunks = rows // _CHUNK_ROWS
    a2 = sr.reshape(rows, pw)
    b2 = hr.reshape(rows, pw)

    out = pl.pallas_call(
        functools.partial(_absdiff_manual, scale=scale, nchunks=nchunks),
        in_specs=[
            pl.BlockSpec(memory_space=pl.ANY),
            pl.BlockSpec(memory_space=pl.ANY),
        ],
        out_specs=pl.BlockSpec(memory_space=pltpu.SMEM),
        out_shape=jax.ShapeDtypeStruct((1, 1), jnp.float32),
        scratch_shapes=[
            pltpu.VMEM((_NBUF, _CHUNK_ROWS, pw), jnp.float32),
            pltpu.VMEM((_NBUF, _CHUNK_ROWS, pw), jnp.float32),
            pltpu.SemaphoreType.DMA((_NBUF,)),
            pltpu.SemaphoreType.DMA((_NBUF,)),
        ],
    )(a2, b2)
    return out[0, 0]
